# baseline (speedup 1.0000x reference)
import functools

import jax
import jax.numpy as jnp
from jax import lax
from jax.experimental import pallas as pl
from jax.experimental.pallas import tpu as pltpu
from jax.experimental.pallas import tpu_sc as plsc

N = 100000
D = 128
N_TYPES = 4
L = 16
NC, NS = 2, 16
NW = NC * NS
CHUNK = 128
NFULL = N // CHUNK
TAIL = N - NFULL * CHUNK
ITERS = (NFULL + NW - 1) // NW
TAIL_WID = NFULL % NW


def _permute(x, idx):
    dnums = lax.GatherDimensionNumbers(
        offset_dims=(), collapsed_slice_dims=(0,), start_index_map=(0,))
    return lax.gather(x, idx[:, None], dnums, slice_sizes=(1,),
                      mode=lax.GatherScatterMode.PROMISE_IN_BOUNDS)


def _body(desc_hbm, elems_hbm, wt_hbm, out_hbm, dbuf, ebuf, wbuf, obuf):
    wid = lax.axis_index("s") * NC + lax.axis_index("c")
    pltpu.sync_copy(wt_hbm, wbuf)
    lane = lax.broadcasted_iota(jnp.int32, (L,), 0)

    def do_chunk(base, rows):
        pltpu.sync_copy(desc_hbm.at[pl.ds(base, rows), :],
                        dbuf.at[pl.ds(0, rows), :])
        pltpu.sync_copy(elems_hbm.at[pl.ds(base, rows)],
                        ebuf.at[pl.ds(0, rows)])

        @plsc.parallel_loop(0, rows // L, unroll=2)
        def group_body(g):
            ev = ebuf[pl.ds(g * L, L)]
            res = jnp.zeros((L,), jnp.float32)
            for k in range(L):
                e = ev[k]
                r = g * L + k
                acc0 = dbuf[r, pl.ds(0, L)] * wbuf[e, pl.ds(0, L)]
                acc1 = dbuf[r, pl.ds(L, L)] * wbuf[e, pl.ds(L, L)]
                for j in range(2, D // L, 2):
                    acc0 = acc0 + dbuf[r, pl.ds(j * L, L)] * wbuf[e, pl.ds(j * L, L)]
                    acc1 = acc1 + dbuf[r, pl.ds((j + 1) * L, L)] * wbuf[e, pl.ds((j + 1) * L, L)]
                acc = acc0 + acc1
                for sh in (8, 4, 2, 1):
                    acc = acc + _permute(acc, lane ^ sh)
                res = jnp.where(lane == k, acc, res)
            obuf[pl.ds(g * L, L)] = res
        pltpu.sync_copy(obuf.at[pl.ds(0, rows)],
                        out_hbm.at[pl.ds(base, rows)])

    def iter_body(i, _):
        c = i * NW + wid

        @pl.when(c < NFULL)
        def _():
            do_chunk(c * CHUNK, CHUNK)

        return 0

    lax.fori_loop(0, ITERS, iter_body, 0)

    @pl.when(wid == TAIL_WID)
    def _():
        do_chunk(NFULL * CHUNK, TAIL)


_mesh = plsc.VectorSubcoreMesh(core_axis_name="c", subcore_axis_name="s")

_sc_dot = functools.partial(
    pl.kernel,
    mesh=_mesh,
    out_type=jax.ShapeDtypeStruct((N,), jnp.float32),
    scratch_types=[
        pltpu.VMEM((CHUNK, D), jnp.float32),
        pltpu.VMEM((CHUNK,), jnp.int32),
        pltpu.VMEM((N_TYPES, D), jnp.float32),
        pltpu.VMEM((CHUNK,), jnp.float32),
    ],
)(_body)


def kernel(descriptors, elems, W, b):
    wt = W.reshape(N_TYPES, D)
    dots = _sc_dot(descriptors, elems.astype(jnp.int32), wt)
    return dots.reshape(N, 1) + b


# speedup vs baseline: 1.2478x; 1.2478x over previous
import functools

import jax
import jax.numpy as jnp
from jax import lax
from jax.experimental import pallas as pl
from jax.experimental.pallas import tpu as pltpu
from jax.experimental.pallas import tpu_sc as plsc

N = 100000
D = 128
N_TYPES = 4
L = 16
NC, NS = 2, 16
NW = NC * NS
CHUNK = 256
NFULL = N // CHUNK
TAIL = N - NFULL * CHUNK
ITERS = (NFULL + NW - 1) // NW
TAIL_WID = NFULL % NW
PAIRS = (ITERS - 1) // 2
LAST_I = 2 * PAIRS


def _permute(x, idx):
    dnums = lax.GatherDimensionNumbers(
        offset_dims=(), collapsed_slice_dims=(0,), start_index_map=(0,))
    return lax.gather(x, idx[:, None], dnums, slice_sizes=(1,),
                      mode=lax.GatherScatterMode.PROMISE_IN_BOUNDS)


def _body(desc_hbm, elems_hbm, wt_hbm, out_hbm,
          dbuf0, dbuf1, ebuf0, ebuf1, wbuf, obuf0, obuf1,
          isem0, isem1, osem0, osem1):
    wid = lax.axis_index("s") * NC + lax.axis_index("c")
    dbuf = (dbuf0, dbuf1)
    ebuf = (ebuf0, ebuf1)
    obuf = (obuf0, obuf1)
    isem = (isem0, isem1)
    osem = (osem0, osem1)
    pltpu.sync_copy(wt_hbm, wbuf)
    lane = lax.broadcasted_iota(jnp.int32, (L,), 0)

    def start_in(b, c):
        pltpu.async_copy(desc_hbm.at[pl.ds(c * CHUNK, CHUNK), :], dbuf[b],
                         isem[b])
        pltpu.async_copy(elems_hbm.at[pl.ds(c * CHUNK, CHUNK)], ebuf[b],
                         isem[b])

    def wait_in(b):
        pltpu.make_async_copy(desc_hbm.at[pl.ds(0, CHUNK), :], dbuf[b],
                              isem[b]).wait()
        pltpu.make_async_copy(elems_hbm.at[pl.ds(0, CHUNK)], ebuf[b],
                              isem[b]).wait()

    def start_out(b, c):
        pltpu.async_copy(obuf[b], out_hbm.at[pl.ds(c * CHUNK, CHUNK)],
                         osem[b])

    def wait_out(b):
        pltpu.make_async_copy(obuf[b], out_hbm.at[pl.ds(0, CHUNK)],
                              osem[b]).wait()

    def compute(b, rows):
        db, eb, ob = dbuf[b], ebuf[b], obuf[b]

        @plsc.parallel_loop(0, rows // L, unroll=2)
        def group_body(g):
            ev = eb[pl.ds(g * L, L)]
            res = jnp.zeros((L,), jnp.float32)
            for k in range(L):
                e = ev[k]
                r = g * L + k
                acc0 = db[r, pl.ds(0, L)] * wbuf[e, pl.ds(0, L)]
                acc1 = db[r, pl.ds(L, L)] * wbuf[e, pl.ds(L, L)]
                for j in range(2, D // L, 2):
                    acc0 = acc0 + db[r, pl.ds(j * L, L)] * wbuf[e, pl.ds(j * L, L)]
                    acc1 = acc1 + db[r, pl.ds((j + 1) * L, L)] * wbuf[e, pl.ds((j + 1) * L, L)]
                acc = acc0 + acc1
                for sh in (8, 4, 2, 1):
                    acc = acc + _permute(acc, lane ^ sh)
                res = jnp.where(lane == k, acc, res)
            ob[pl.ds(g * L, L)] = res

    start_in(0, wid)
    start_in(1, NW + wid)

    def pair_body(p, _):
        for b in (0, 1):
            i = 2 * p + b
            c = i * NW + wid
            cn2 = c + 2 * NW
            wait_in(b)

            @pl.when(p >= 1)
            def _():
                wait_out(b)

            compute(b, CHUNK)
            start_out(b, c)

            @pl.when(cn2 < NFULL)
            def _():
                start_in(b, cn2)

        return 0

    lax.fori_loop(0, PAIRS, pair_body, 0)

    c_last = LAST_I * NW + wid

    @pl.when(c_last < NFULL)
    def _():
        wait_in(0)
        wait_out(0)
        compute(0, CHUNK)
        start_out(0, c_last)

    @pl.when(wid == TAIL_WID)
    def _():
        wait_out(0)
        base = NFULL * CHUNK
        pltpu.sync_copy(desc_hbm.at[pl.ds(base, TAIL), :],
                        dbuf0.at[pl.ds(0, TAIL), :])
        pltpu.sync_copy(elems_hbm.at[pl.ds(base, TAIL)],
                        ebuf0.at[pl.ds(0, TAIL)])
        compute(0, TAIL)
        pltpu.sync_copy(obuf0.at[pl.ds(0, TAIL)],
                        out_hbm.at[pl.ds(base, TAIL)])

    @pl.when(wid != TAIL_WID)
    def _():
        wait_out(0)

    wait_out(1)


_mesh = plsc.VectorSubcoreMesh(core_axis_name="c", subcore_axis_name="s")

_sc_dot = functools.partial(
    pl.kernel,
    mesh=_mesh,
    out_type=jax.ShapeDtypeStruct((N,), jnp.float32),
    scratch_types=[
        pltpu.VMEM((CHUNK, D), jnp.float32),
        pltpu.VMEM((CHUNK, D), jnp.float32),
        pltpu.VMEM((CHUNK,), jnp.int32),
        pltpu.VMEM((CHUNK,), jnp.int32),
        pltpu.VMEM((N_TYPES, D), jnp.float32),
        pltpu.VMEM((CHUNK,), jnp.float32),
        pltpu.VMEM((CHUNK,), jnp.float32),
        pltpu.SemaphoreType.DMA,
        pltpu.SemaphoreType.DMA,
        pltpu.SemaphoreType.DMA,
        pltpu.SemaphoreType.DMA,
    ],
)(_body)


def kernel(descriptors, elems, W, b):
    wt = W.reshape(N_TYPES, D)
    dots = _sc_dot(descriptors, elems.astype(jnp.int32), wt)
    return dots.reshape(N, 1) + b
